# baseline (speedup 1.0000x reference)
import jax
import jax.numpy as jnp
from jax import lax
from jax.experimental import pallas as pl
from jax.experimental.pallas import tpu as pltpu
from jax.experimental.pallas import tpu_sc as plsc

D_MODEL = 128
N_NODES = 320000
N_GRAPHS = 512
_HALF = N_NODES // 2


_BN = 32000


def _matvec_body(x_ref, w_ref, b_ref, o_ref):
    s = lax.dot_general(
        w_ref[...], x_ref[...],
        (((0,), (1,)), ((), ())),
        preferred_element_type=jnp.float32,
    )
    o_ref[...] = (s + b_ref[0, 0])[None]


def _node_scalars(x, W, b):
    nb = x.shape[0] // _BN
    return pl.pallas_call(
        _matvec_body,
        grid=(nb,),
        in_specs=[
            pl.BlockSpec((_BN, D_MODEL), lambda i: (i, 0)),
            pl.BlockSpec((D_MODEL, 1), lambda i: (0, 0)),
            pl.BlockSpec((1, 1), lambda i: (0, 0)),
        ],
        out_specs=pl.BlockSpec((1, 1, _BN), lambda i: (i, 0, 0)),
        out_shape=jax.ShapeDtypeStruct((nb, 1, _BN), jnp.float32),
    )(x, W, b.reshape(1, 1)).reshape(x.shape[0])


_NT = 16
_TBL = N_GRAPHS * 16
_SEG_PER_TILE = N_GRAPHS // _NT


def _make_segsum(n, with_partial):
    chunk = n // _NT
    nv = chunk // 16
    unroll = 5
    assert nv % unroll == 0

    def body(*refs):
        if with_partial:
            (s_hbm, ids_hbm, part_hbm, out_hbm, vals_v, ids_v, tbl_v, buf_v,
             acc_v, out_v, part_v, shared, sem_a, sem_b) = refs
        else:
            (s_hbm, ids_hbm, out_hbm, vals_v, ids_v, tbl_v, buf_v,
             acc_v, out_v, part_v, shared, sem_a, sem_b) = refs
        tid = lax.axis_index("s")
        lane = lax.iota(jnp.int32, 16)
        seg0 = tid * _SEG_PER_TILE

        base = tid * chunk
        h_a = pltpu.async_copy(s_hbm.at[pl.ds(base, chunk)], vals_v, sem_a)
        h_b = pltpu.async_copy(ids_hbm.at[pl.ds(base, chunk)], ids_v, sem_b)
        if with_partial:
            pltpu.sync_copy(part_hbm.at[pl.ds(seg0, _SEG_PER_TILE)], part_v)

        zeros = jnp.zeros((16,), jnp.float32)

        def _zero(i, c):
            tbl_v[pl.ds(i * 16, 16)] = zeros
            return c

        lax.fori_loop(0, _TBL // 16, _zero, 0)
        h_a.wait()
        h_b.wait()

        def _scat(i, c):
            for u in range(unroll):
                sl = pl.ds((i * unroll + u) * 16, 16)
                plsc.addupdate_scatter(
                    tbl_v, [ids_v[sl] * 16 + lane], vals_v[sl])
            return c

        lax.fori_loop(0, nv // unroll, _scat, 0)

        pltpu.sync_copy(tbl_v, shared.at[tid])
        plsc.subcore_barrier()

        def _zacc(i, c):
            acc_v[pl.ds(i * 16, 16)] = zeros
            return c

        lax.fori_loop(0, (_SEG_PER_TILE * 16) // 16, _zacc, 0)

        for src in range(_NT):
            pltpu.sync_copy(
                shared.at[src, pl.ds(seg0 * 16, _SEG_PER_TILE * 16)], buf_v)
            for i in range((_SEG_PER_TILE * 16) // 16):
                sl = pl.ds(i * 16, 16)
                acc_v[sl] = acc_v[sl] + buf_v[sl]

        for v in range(_SEG_PER_TILE // 16):
            sl = pl.ds(v * 16, 16)
            r = part_v[sl] if with_partial else jnp.zeros((16,), jnp.float32)
            for c in range(16):
                r = r + plsc.load_gather(acc_v, [(lane + v * 16) * 16 + c])
            out_v[sl] = r

        pltpu.sync_copy(out_v, out_hbm.at[pl.ds(seg0, _SEG_PER_TILE)])

    mesh = plsc.VectorSubcoreMesh(
        core_axis_name="c", subcore_axis_name="s", num_cores=1)
    return pl.kernel(
        body,
        out_type=jax.ShapeDtypeStruct((N_GRAPHS,), jnp.float32),
        mesh=mesh,
        compiler_params=pltpu.CompilerParams(needs_layout_passes=False),
        scratch_types=[
            pltpu.VMEM((chunk,), jnp.float32),
            pltpu.VMEM((chunk,), jnp.int32),
            pltpu.VMEM((_TBL,), jnp.float32),
            pltpu.VMEM((_SEG_PER_TILE * 16,), jnp.float32),
            pltpu.VMEM((_SEG_PER_TILE * 16,), jnp.float32),
            pltpu.VMEM((_SEG_PER_TILE,), jnp.float32),
            pltpu.VMEM((_SEG_PER_TILE,), jnp.float32),
            pltpu.VMEM_SHARED((_NT, _TBL), jnp.float32),
            pltpu.SemaphoreType.DMA,
            pltpu.SemaphoreType.DMA,
        ],
    )


def kernel(x, batch, W, b):
    ids = batch.astype(jnp.int32)
    s_a = _node_scalars(x[:_HALF], W, b)
    part = _make_segsum(_HALF, False)(s_a, ids[:_HALF])
    s_b = _node_scalars(x[_HALF:], W, b)
    return _make_segsum(_HALF, True)(s_b, ids[_HALF:], part)


# speedup vs baseline: 2.1805x; 2.1805x over previous
import jax
import jax.numpy as jnp
from jax import lax
from jax.experimental import pallas as pl
from jax.experimental.pallas import tpu as pltpu
from jax.experimental.pallas import tpu_sc as plsc

D_MODEL = 128
N_NODES = 320000
N_GRAPHS = 512
_HALF = N_NODES // 2


_BN = 20000
_NB = _HALF // _BN


def _matvec_body(xa_ref, xb_ref, w_ref, b_ref, o_ref):
    dims = (((0,), (1,)), ((), ()))
    sa = lax.dot_general(w_ref[...], xa_ref[...], dims,
                         preferred_element_type=jnp.float32)
    sb = lax.dot_general(w_ref[...], xb_ref[...], dims,
                         preferred_element_type=jnp.float32)
    o_ref[...] = (jnp.concatenate([sa, sb], axis=0) + b_ref[0, 0])[None]


def _node_scalars(x, W, b):
    return pl.pallas_call(
        _matvec_body,
        grid=(_NB,),
        in_specs=[
            pl.BlockSpec((_BN, D_MODEL), lambda i: (i, 0)),
            pl.BlockSpec((_BN, D_MODEL), lambda i: (i + _NB, 0)),
            pl.BlockSpec((D_MODEL, 1), lambda i: (0, 0)),
            pl.BlockSpec((1, 1), lambda i: (0, 0)),
        ],
        out_specs=pl.BlockSpec((1, 2, _BN), lambda i: (i, 0, 0)),
        out_shape=jax.ShapeDtypeStruct((_NB, 2, _BN), jnp.float32),
    )(x, x, W, b.reshape(1, 1)).reshape(N_NODES)


_NT = 16
_TBL = N_GRAPHS * 16
_SEG_PER_TILE = N_GRAPHS // _NT
_CHUNK = N_NODES // _NT
_NV = _CHUNK // 16
_UNROLL = 10


def _segsum_body(s_hbm, ids_hbm, out_hbm, vals_v, ids_v, tbl_v, buf_v,
                 acc_v, out_v, shared, sem_a, sem_b):
    tid = lax.axis_index("s")
    lane = lax.iota(jnp.int32, 16)
    seg0 = tid * _SEG_PER_TILE

    base = tid * _CHUNK
    ids_base = (tid % 2) * _HALF + (tid // 2) * _CHUNK
    h_a = pltpu.async_copy(s_hbm.at[pl.ds(base, _CHUNK)], vals_v, sem_a)
    h_b = pltpu.async_copy(ids_hbm.at[pl.ds(ids_base, _CHUNK)], ids_v, sem_b)

    zeros = jnp.zeros((16,), jnp.float32)

    def _zero(i, c):
        tbl_v[pl.ds(i * 16, 16)] = zeros
        return c

    lax.fori_loop(0, _TBL // 16, _zero, 0)
    h_a.wait()
    h_b.wait()

    def _scat(i, c):
        for u in range(_UNROLL):
            sl = pl.ds((i * _UNROLL + u) * 16, 16)
            plsc.addupdate_scatter(tbl_v, [ids_v[sl] * 16 + lane], vals_v[sl])
        return c

    lax.fori_loop(0, _NV // _UNROLL, _scat, 0)

    pltpu.sync_copy(tbl_v, shared.at[tid])
    plsc.subcore_barrier()

    def _zacc(i, c):
        acc_v[pl.ds(i * 16, 16)] = zeros
        return c

    lax.fori_loop(0, (_SEG_PER_TILE * 16) // 16, _zacc, 0)

    for src in range(_NT):
        pltpu.sync_copy(
            shared.at[src, pl.ds(seg0 * 16, _SEG_PER_TILE * 16)], buf_v)
        for i in range((_SEG_PER_TILE * 16) // 16):
            sl = pl.ds(i * 16, 16)
            acc_v[sl] = acc_v[sl] + buf_v[sl]

    for v in range(_SEG_PER_TILE // 16):
        r = jnp.zeros((16,), jnp.float32)
        for c in range(16):
            r = r + plsc.load_gather(acc_v, [(lane + v * 16) * 16 + c])
        out_v[pl.ds(v * 16, 16)] = r

    pltpu.sync_copy(out_v, out_hbm.at[pl.ds(seg0, _SEG_PER_TILE)])


def _segment_sum(s, ids):
    mesh = plsc.VectorSubcoreMesh(
        core_axis_name="c", subcore_axis_name="s", num_cores=1)
    return pl.kernel(
        _segsum_body,
        out_type=jax.ShapeDtypeStruct((N_GRAPHS,), jnp.float32),
        mesh=mesh,
        compiler_params=pltpu.CompilerParams(needs_layout_passes=False),
        scratch_types=[
            pltpu.VMEM((_CHUNK,), jnp.float32),
            pltpu.VMEM((_CHUNK,), jnp.int32),
            pltpu.VMEM((_TBL,), jnp.float32),
            pltpu.VMEM((_SEG_PER_TILE * 16,), jnp.float32),
            pltpu.VMEM((_SEG_PER_TILE * 16,), jnp.float32),
            pltpu.VMEM((_SEG_PER_TILE,), jnp.float32),
            pltpu.VMEM_SHARED((_NT, _TBL), jnp.float32),
            pltpu.SemaphoreType.DMA,
            pltpu.SemaphoreType.DMA,
        ],
    )(s, ids)


def kernel(x, batch, W, b):
    s = _node_scalars(x, W, b)
    ids = batch.astype(jnp.int32)
    return _segment_sum(s, ids)
